# baseline (speedup 1.0000x reference)
import functools

import jax
import jax.numpy as jnp
from jax import lax
from jax.experimental import pallas as pl
from jax.experimental.pallas import tpu as pltpu
from jax.experimental.pallas import tpu_sc as plsc

N = 10000
E = 320000
DN = 128
DE = 16
G = 128

NC = 2
NS = 16
NW = NC * NS
EH = E
EPW = EH // NW
C = 80
NCH = EPW // C
RPT = 624
TAIL = N - NS * RPT

EB = 10000
NEB = EH // EB
NB = 2000
NNB = N // NB


def _lrelu(v):
    return jnp.where(v >= 0, v, 0.01 * v)


def _tc0_body(x_ref, wnT, bn, we1aT, be1, w2a, be2, hv_ref, xw1p_ref, sn_ref):
    x = x_ref[...]
    hv = _lrelu(jnp.dot(x, wnT[...], preferred_element_type=jnp.float32) + bn[...])
    hv_ref[...] = hv
    xw1p_ref[...] = jnp.dot(x, we1aT[...], preferred_element_type=jnp.float32) + be1[...]
    sn_ref[...] = jnp.dot(hv, w2a[...], preferred_element_type=jnp.float32) + be2[...]


def _tc0(x, wnT, bn2, we1aT, be12, w2a, be22):
    return pl.pallas_call(
        _tc0_body,
        out_shape=(
            jax.ShapeDtypeStruct((N, G), jnp.float32),
            jax.ShapeDtypeStruct((N, G), jnp.float32),
            jax.ShapeDtypeStruct((N, 1), jnp.float32),
        ),
    )(x, wnT, bn2, we1aT, be12, w2a, be22)


_MESH = plsc.VectorSubcoreMesh(core_axis_name="c", subcore_axis_name="s")


@functools.partial(
    pl.kernel,
    mesh=_MESH,
    out_type=[
        jax.ShapeDtypeStruct((EH, G), jnp.float32),
        jax.ShapeDtypeStruct((NW, NCH, C), jnp.float32),
    ],
    scratch_types=[
        pltpu.VMEM((NCH, C), jnp.int32),
        pltpu.VMEM((NCH, C), jnp.int32),
        pltpu.VMEM((C, G), jnp.float32),
        pltpu.VMEM((C, G), jnp.float32),
        pltpu.VMEM((C, G), jnp.float32),
        pltpu.VMEM((C, G), jnp.float32),
        pltpu.VMEM((C, G), jnp.float32),
        pltpu.VMEM((NCH, C), jnp.float32),
        pltpu.SemaphoreType.DMA,
        pltpu.SemaphoreType.DMA,
        pltpu.SemaphoreType.DMA,
        pltpu.SemaphoreType.DMA,
        pltpu.SemaphoreType.DMA,
        pltpu.SemaphoreType.DMA,
        pltpu.SemaphoreType.DMA,
        pltpu.SemaphoreType.DMA,
        pltpu.SemaphoreType.DMA,
        pltpu.SemaphoreType.DMA,
        pltpu.SemaphoreType.DMA,
    ],
)
def _sc1(xw1p_hbm, snode_hbm, src_hbm, dst_hbm, g1_hbm, sdst_hbm,
         idxs_v, idxd_v, rows0, rows1, rows2, rows3, rows4, svals,
         gs0, gs1, gs2, gs3, gs4, ws0, ws1, ws2, ws3, ws4, ssem):
    w = lax.axis_index("s") * NC + lax.axis_index("c")
    pltpu.sync_copy(src_hbm.at[w], idxs_v)
    pltpu.sync_copy(dst_hbm.at[w], idxd_v)

    base = w * EPW
    rows = (rows0, rows1, rows2, rows3, rows4)
    gss = (gs0, gs1, gs2, gs3, gs4)
    wss = (ws0, ws1, ws2, ws3, ws4)

    def g_start(j, b):
        pltpu.async_copy(xw1p_hbm.at[idxs_v.at[j]], rows[b], gss[b])

    def g_wait(b):
        pltpu.make_async_copy(xw1p_hbm.at[idxs_v.at[0]], rows[b], gss[b]).wait()

    def w_start(j, b):
        pltpu.async_copy(rows[b], g1_hbm.at[pl.ds(base + j * C, C)], wss[b])

    def w_wait(b):
        pltpu.make_async_copy(rows[b], g1_hbm.at[pl.ds(base, C)], wss[b]).wait()

    def s_start(j):
        pltpu.async_copy(snode_hbm.at[idxd_v.at[j]], svals.at[j], ssem)

    def s_wait():
        pltpu.make_async_copy(snode_hbm.at[idxd_v.at[0]], svals.at[0], ssem).wait()

    for b in range(5):
        g_start(b, b)

    def body(i, carry):
        j = 5 * i
        for b in range(5):
            s_start(j + b)
        for b in range(5):
            g_wait(b)
            w_start(j + b, b)
        for b in range(5):
            w_wait(b)
            g_start(j + 5 + b, b)
        for b in range(5):
            s_wait()
        return carry

    lax.fori_loop(0, NCH // 5 - 1, body, 0)
    j = NCH - 5
    for b in range(5):
        s_start(j + b)
    for b in range(5):
        g_wait(b)
        w_start(j + b, b)
    for b in range(5):
        w_wait(b)
    for b in range(5):
        s_wait()
    pltpu.sync_copy(svals, sdst_hbm.at[w])


def _tc1_body(g1_ref, ea_ref, sd_ref, we1bT, w2bc, hx_ref, ex_ref):
    z = g1_ref[...] + jnp.dot(ea_ref[...], we1bT[...],
                              preferred_element_type=jnp.float32)
    h = _lrelu(z)
    t = jnp.dot(h, w2bc[...], preferred_element_type=jnp.float32)[:, 0] \
        + sd_ref[0, 0, :]
    ex = jnp.exp(_lrelu(t))
    hx_ref[...] = h * ex[:, None]
    ex_ref[0, 0, :] = ex


def _tc1(g1, ea, sd2, we1bT, w2b):
    return pl.pallas_call(
        _tc1_body,
        grid=(NEB,),
        in_specs=[
            pl.BlockSpec((EB, G), lambda i: (i, 0)),
            pl.BlockSpec((EB, DE), lambda i: (i, 0)),
            pl.BlockSpec((1, 1, EB), lambda i: (i, 0, 0)),
            pl.BlockSpec((DE, G), lambda i: (0, 0)),
            pl.BlockSpec((G, 1), lambda i: (0, 0)),
        ],
        out_specs=[
            pl.BlockSpec((EB, G), lambda i: (i, 0)),
            pl.BlockSpec((1, 1, EB), lambda i: (i, 0, 0)),
        ],
        out_shape=(
            jax.ShapeDtypeStruct((EH, G), jnp.float32),
            jax.ShapeDtypeStruct((NEB, 1, EB), jnp.float32),
        ),
    )(g1, ea, sd2, we1bT, w2b)


@functools.partial(
    pl.kernel,
    mesh=_MESH,
    out_type=[
        jax.ShapeDtypeStruct((NC, N, G), jnp.float32),
        jax.ShapeDtypeStruct((N,), jnp.float32),
        jax.ShapeDtypeStruct((N,), jnp.float32),
    ],
    scratch_types=[
        pltpu.VMEM((NCH, C), jnp.int32),
        pltpu.VMEM((1, C), jnp.float32),
        pltpu.VMEM((1, C), jnp.float32),
        pltpu.VMEM((1, C), jnp.float32),
        pltpu.VMEM((C, G), jnp.float32),
        pltpu.VMEM((C, G), jnp.float32),
        pltpu.VMEM((C, G), jnp.float32),
        pltpu.VMEM_SHARED((N, G), jnp.float32),
        pltpu.VMEM_SHARED((N,), jnp.float32),
        pltpu.SemaphoreType.DMA,
        pltpu.SemaphoreType.DMA,
        pltpu.SemaphoreType.DMA,
    ],
)
def _sc2(hx_hbm, ex_hbm, dst_hbm, zc_hbm, zs_hbm, cpart_hbm, s0_hbm, s1_hbm,
         idxd_v, exr0, exr1, exr2, rows0, rows1, rows2, cacc, sacc,
         rs0, rs1, rs2):
    c = lax.axis_index("c")
    s = lax.axis_index("s")
    w = s * NC + c
    pltpu.sync_copy(zc_hbm.at[pl.ds(s * RPT, RPT)], cacc.at[pl.ds(s * RPT, RPT)])

    @pl.when(s == NS - 1)
    def _zero_tail():
        pltpu.sync_copy(zc_hbm.at[pl.ds(NS * RPT, TAIL)],
                        cacc.at[pl.ds(NS * RPT, TAIL)])

    @pl.when(s == 0)
    def _zero_s():
        pltpu.sync_copy(zs_hbm, sacc)

    plsc.subcore_barrier()

    pltpu.sync_copy(dst_hbm.at[w], idxd_v)
    base = w * EPW
    wch = w * NCH
    rows = (rows0, rows1, rows2)
    exrs = (exr0, exr1, exr2)
    rss = (rs0, rs1, rs2)

    def r_start(j, b):
        pltpu.async_copy(hx_hbm.at[pl.ds(base + j * C, C)], rows[b], rss[b])
        pltpu.async_copy(ex_hbm.at[wch + j], exrs[b], rss[b])

    def r_wait(b):
        pltpu.make_async_copy(hx_hbm.at[pl.ds(base, C)], rows[b], rss[b]).wait()
        pltpu.make_async_copy(ex_hbm.at[wch], exrs[b], rss[b]).wait()

    def scat(j, b):
        pltpu.sync_copy(rows[b], cacc.at[idxd_v.at[j]], add=True)
        pltpu.sync_copy(exrs[b].at[0], sacc.at[idxd_v.at[j]], add=True)

    for b in range(3):
        r_start(b, b)

    def body(i, carry):
        j = 3 * i
        for b in range(3):
            r_wait(b)
            scat(j + b, b)
            r_start(j + 3 + b, b)
        return carry

    lax.fori_loop(0, NCH // 3 - 1, body, 0)
    j = NCH - 5
    r_wait(0)
    scat(j, 0)
    r_start(j + 3, 0)
    r_wait(1)
    scat(j + 1, 1)
    r_start(j + 4, 1)
    r_wait(2)
    scat(j + 2, 2)
    r_wait(0)
    scat(j + 3, 0)
    r_wait(1)
    scat(j + 4, 1)
    plsc.subcore_barrier()
    pltpu.sync_copy(cacc.at[pl.ds(s * RPT, RPT)],
                    cpart_hbm.at[c].at[pl.ds(s * RPT, RPT)])

    @pl.when(s == NS - 1)
    def _out_tail():
        pltpu.sync_copy(cacc.at[pl.ds(NS * RPT, TAIL)],
                        cpart_hbm.at[c].at[pl.ds(NS * RPT, TAIL)])

    @pl.when(jnp.logical_and(s == 0, c == 0))
    def _out_s0():
        pltpu.sync_copy(sacc, s0_hbm)

    @pl.when(jnp.logical_and(s == 0, c == 1))
    def _out_s1():
        pltpu.sync_copy(sacc, s1_hbm)


def _tc2_body(cp_ref, sp_ref, hv_ref, wtT, bt2, wihT, bih2, whhT, bhh2, out_ref):
    cs = cp_ref[0] + cp_ref[1]
    sv = sp_ref[0, :, 0] + sp_ref[1, :, 0]
    inv = 1.0 / (sv + 1e-16)
    cmean = cs * inv[:, None]
    cfull = jnp.dot(cmean, wtT[...], preferred_element_type=jnp.float32) \
        + (sv * inv)[:, None] * bt2[...]
    ctx = jnp.where(cfull > 0, cfull, jnp.exp(cfull) - 1.0)
    hv = hv_ref[...]
    gi = jnp.dot(ctx, wihT[...], preferred_element_type=jnp.float32) + bih2[...]
    gh = jnp.dot(hv, whhT[...], preferred_element_type=jnp.float32) + bhh2[...]
    r = jax.nn.sigmoid(gi[:, :G] + gh[:, :G])
    zz = jax.nn.sigmoid(gi[:, G:2 * G] + gh[:, G:2 * G])
    nn = jnp.tanh(gi[:, 2 * G:] + r * gh[:, 2 * G:])
    out_ref[...] = jnp.maximum((1.0 - zz) * nn + zz * hv, 0.0)


def _tc2(cpa, sp3, hv, wtT, bt2, wihT, bih2, whhT, bhh2):
    return pl.pallas_call(
        _tc2_body,
        grid=(NNB,),
        in_specs=[
            pl.BlockSpec((NC, NB, G), lambda i: (0, i, 0)),
            pl.BlockSpec((NC, NB, 1), lambda i: (0, i, 0)),
            pl.BlockSpec((NB, G), lambda i: (i, 0)),
            pl.BlockSpec((G, G), lambda i: (0, 0)),
            pl.BlockSpec((1, G), lambda i: (0, 0)),
            pl.BlockSpec((G, 3 * G), lambda i: (0, 0)),
            pl.BlockSpec((1, 3 * G), lambda i: (0, 0)),
            pl.BlockSpec((G, 3 * G), lambda i: (0, 0)),
            pl.BlockSpec((1, 3 * G), lambda i: (0, 0)),
        ],
        out_specs=pl.BlockSpec((NB, G), lambda i: (i, 0)),
        out_shape=jax.ShapeDtypeStruct((N, G), jnp.float32),
    )(cpa, sp3, hv, wtT, bt2, wihT, bih2, whhT, bhh2)


def kernel(x, edge_index, edge_attr, Wn, bn, We1, be1, We2, be2, Wt, bt,
           Wih, bih, Whh, bhh):
    src = edge_index[0]
    dst = edge_index[1]
    wnT = Wn.T
    we1aT = We1[:, :DN].T
    we1bT = We1[:, DN:].T
    w2a = We2[0, :G].reshape(G, 1)
    w2b = We2[0, G:].reshape(G, 1)

    hv, xw1p, snode = _tc0(x, wnT, bn.reshape(1, G), we1aT, be1.reshape(1, G),
                           w2a, be2.reshape(1, 1))

    sn1 = snode.reshape(N)
    zc = jnp.zeros((N, G), jnp.float32)
    zs = jnp.zeros((N,), jnp.float32)

    src2 = src.reshape(NW, NCH, C)
    dst2 = dst.reshape(NW, NCH, C)

    g1, sdst = _sc1(xw1p, sn1, src2, dst2)
    hx, ex3 = _tc1(g1, edge_attr, sdst.reshape(NEB, 1, EB), we1bT, w2b)
    cpart, s0, s1 = _sc2(hx, ex3.reshape(NW * NCH, 1, C), dst2, zc, zs)
    spart = jnp.stack([s0, s1]).reshape(NC, N, 1)

    return _tc2(cpart, spart, hv, Wt.T, bt.reshape(1, G),
                Wih.T, bih.reshape(1, 3 * G), Whh.T, bhh.reshape(1, 3 * G))


# speedup vs baseline: 1.0024x; 1.0024x over previous
import functools

import jax
import jax.numpy as jnp
from jax import lax
from jax.experimental import pallas as pl
from jax.experimental.pallas import tpu as pltpu
from jax.experimental.pallas import tpu_sc as plsc

N = 10000
E = 320000
DN = 128
DE = 16
G = 128

NC = 2
NS = 16
NW = NC * NS
EH = E
EPW = EH // NW
C = 80
NCH = EPW // C
RPT = 624
TAIL = N - NS * RPT

EB = 8000
NEB = EH // EB
NB = 2000
NNB = N // NB


def _lrelu(v):
    return jnp.where(v >= 0, v, 0.01 * v)


def _tc0_body(x_ref, wnT, bn, we1aT, be1, w2a, be2, hv_ref, xw1p_ref, sn_ref):
    x = x_ref[...]
    hv = _lrelu(jnp.dot(x, wnT[...], preferred_element_type=jnp.float32) + bn[...])
    hv_ref[...] = hv
    xw1p_ref[...] = jnp.dot(x, we1aT[...], preferred_element_type=jnp.float32) + be1[...]
    sn_ref[...] = jnp.dot(hv, w2a[...], preferred_element_type=jnp.float32) + be2[...]


def _tc0(x, wnT, bn2, we1aT, be12, w2a, be22):
    return pl.pallas_call(
        _tc0_body,
        out_shape=(
            jax.ShapeDtypeStruct((N, G), jnp.float32),
            jax.ShapeDtypeStruct((N, G), jnp.float32),
            jax.ShapeDtypeStruct((N, 1), jnp.float32),
        ),
    )(x, wnT, bn2, we1aT, be12, w2a, be22)


_MESH = plsc.VectorSubcoreMesh(core_axis_name="c", subcore_axis_name="s")


@functools.partial(
    pl.kernel,
    mesh=_MESH,
    out_type=[
        jax.ShapeDtypeStruct((EH, G), jnp.float32),
        jax.ShapeDtypeStruct((NW, NCH, C), jnp.float32),
    ],
    scratch_types=[
        pltpu.VMEM((NCH, C), jnp.int32),
        pltpu.VMEM((NCH, C), jnp.int32),
        pltpu.VMEM((C, G), jnp.float32),
        pltpu.VMEM((C, G), jnp.float32),
        pltpu.VMEM((C, G), jnp.float32),
        pltpu.VMEM((C, G), jnp.float32),
        pltpu.VMEM((C, G), jnp.float32),
        pltpu.VMEM((NCH, C), jnp.float32),
        pltpu.SemaphoreType.DMA,
        pltpu.SemaphoreType.DMA,
        pltpu.SemaphoreType.DMA,
        pltpu.SemaphoreType.DMA,
        pltpu.SemaphoreType.DMA,
        pltpu.SemaphoreType.DMA,
        pltpu.SemaphoreType.DMA,
        pltpu.SemaphoreType.DMA,
        pltpu.SemaphoreType.DMA,
        pltpu.SemaphoreType.DMA,
        pltpu.SemaphoreType.DMA,
    ],
)
def _sc1(xw1p_hbm, snode_hbm, src_hbm, dst_hbm, g1_hbm, sdst_hbm,
         idxs_v, idxd_v, rows0, rows1, rows2, rows3, rows4, svals,
         gs0, gs1, gs2, gs3, gs4, ws0, ws1, ws2, ws3, ws4, ssem):
    w = lax.axis_index("s") * NC + lax.axis_index("c")
    pltpu.sync_copy(src_hbm.at[w], idxs_v)
    pltpu.sync_copy(dst_hbm.at[w], idxd_v)

    base = w * EPW
    rows = (rows0, rows1, rows2, rows3, rows4)
    gss = (gs0, gs1, gs2, gs3, gs4)
    wss = (ws0, ws1, ws2, ws3, ws4)

    def g_start(j, b):
        pltpu.async_copy(xw1p_hbm.at[idxs_v.at[j]], rows[b], gss[b])

    def g_wait(b):
        pltpu.make_async_copy(xw1p_hbm.at[idxs_v.at[0]], rows[b], gss[b]).wait()

    def w_start(j, b):
        pltpu.async_copy(rows[b], g1_hbm.at[pl.ds(base + j * C, C)], wss[b])

    def w_wait(b):
        pltpu.make_async_copy(rows[b], g1_hbm.at[pl.ds(base, C)], wss[b]).wait()

    def s_start(j):
        pltpu.async_copy(snode_hbm.at[idxd_v.at[j]], svals.at[j], ssem)

    def s_wait():
        pltpu.make_async_copy(snode_hbm.at[idxd_v.at[0]], svals.at[0], ssem).wait()

    for b in range(5):
        g_start(b, b)

    def body(i, carry):
        j = 5 * i
        for b in range(5):
            s_start(j + b)
        for b in range(5):
            g_wait(b)
            w_start(j + b, b)
        for b in range(5):
            w_wait(b)
            g_start(j + 5 + b, b)
        for b in range(5):
            s_wait()
        return carry

    lax.fori_loop(0, NCH // 5 - 1, body, 0)
    j = NCH - 5
    for b in range(5):
        s_start(j + b)
    for b in range(5):
        g_wait(b)
        w_start(j + b, b)
    for b in range(5):
        w_wait(b)
    for b in range(5):
        s_wait()
    pltpu.sync_copy(svals, sdst_hbm.at[w])


def _tc1_body(g1_ref, ea_ref, sd_ref, we1bT, w2bc, hx_ref, ex_ref):
    z = g1_ref[...] + jnp.dot(ea_ref[...], we1bT[...],
                              preferred_element_type=jnp.float32)
    h = _lrelu(z)
    t = jnp.dot(h, w2bc[...], preferred_element_type=jnp.float32)[:, 0] \
        + sd_ref[0, 0, :]
    ex = jnp.exp(_lrelu(t))
    hx_ref[...] = h * ex[:, None]
    ex_ref[0, 0, :] = ex


def _tc1(g1, ea, sd2, we1bT, w2b):
    return pl.pallas_call(
        _tc1_body,
        grid=(NEB,),
        in_specs=[
            pl.BlockSpec((EB, G), lambda i: (i, 0)),
            pl.BlockSpec((EB, DE), lambda i: (i, 0)),
            pl.BlockSpec((1, 1, EB), lambda i: (i, 0, 0)),
            pl.BlockSpec((DE, G), lambda i: (0, 0)),
            pl.BlockSpec((G, 1), lambda i: (0, 0)),
        ],
        out_specs=[
            pl.BlockSpec((EB, G), lambda i: (i, 0)),
            pl.BlockSpec((1, 1, EB), lambda i: (i, 0, 0)),
        ],
        out_shape=(
            jax.ShapeDtypeStruct((EH, G), jnp.float32),
            jax.ShapeDtypeStruct((NEB, 1, EB), jnp.float32),
        ),
    )(g1, ea, sd2, we1bT, w2b)


@functools.partial(
    pl.kernel,
    mesh=_MESH,
    out_type=[
        jax.ShapeDtypeStruct((NC, N, G), jnp.float32),
        jax.ShapeDtypeStruct((N,), jnp.float32),
        jax.ShapeDtypeStruct((N,), jnp.float32),
    ],
    scratch_types=[
        pltpu.VMEM((NCH, C), jnp.int32),
        pltpu.VMEM((1, C), jnp.float32),
        pltpu.VMEM((1, C), jnp.float32),
        pltpu.VMEM((1, C), jnp.float32),
        pltpu.VMEM((C, G), jnp.float32),
        pltpu.VMEM((C, G), jnp.float32),
        pltpu.VMEM((C, G), jnp.float32),
        pltpu.VMEM_SHARED((N, G), jnp.float32),
        pltpu.VMEM_SHARED((N,), jnp.float32),
        pltpu.SemaphoreType.DMA,
        pltpu.SemaphoreType.DMA,
        pltpu.SemaphoreType.DMA,
    ],
)
def _sc2(hx_hbm, ex_hbm, dst_hbm, zc_hbm, zs_hbm, cpart_hbm, s0_hbm, s1_hbm,
         idxd_v, exr0, exr1, exr2, rows0, rows1, rows2, cacc, sacc,
         rs0, rs1, rs2):
    c = lax.axis_index("c")
    s = lax.axis_index("s")
    w = s * NC + c
    pltpu.sync_copy(zc_hbm.at[pl.ds(s * RPT, RPT)], cacc.at[pl.ds(s * RPT, RPT)])

    @pl.when(s == NS - 1)
    def _zero_tail():
        pltpu.sync_copy(zc_hbm.at[pl.ds(NS * RPT, TAIL)],
                        cacc.at[pl.ds(NS * RPT, TAIL)])

    @pl.when(s == 0)
    def _zero_s():
        pltpu.sync_copy(zs_hbm, sacc)

    plsc.subcore_barrier()

    pltpu.sync_copy(dst_hbm.at[w], idxd_v)
    base = w * EPW
    wch = w * NCH
    rows = (rows0, rows1, rows2)
    exrs = (exr0, exr1, exr2)
    rss = (rs0, rs1, rs2)

    def r_start(j, b):
        pltpu.async_copy(hx_hbm.at[pl.ds(base + j * C, C)], rows[b], rss[b])
        pltpu.async_copy(ex_hbm.at[wch + j], exrs[b], rss[b])

    def r_wait(b):
        pltpu.make_async_copy(hx_hbm.at[pl.ds(base, C)], rows[b], rss[b]).wait()
        pltpu.make_async_copy(ex_hbm.at[wch], exrs[b], rss[b]).wait()

    def scat(j, b):
        pltpu.sync_copy(rows[b], cacc.at[idxd_v.at[j]], add=True)
        pltpu.sync_copy(exrs[b].at[0], sacc.at[idxd_v.at[j]], add=True)

    for b in range(3):
        r_start(b, b)

    def body(i, carry):
        j = 3 * i
        for b in range(3):
            r_wait(b)
            scat(j + b, b)
            r_start(j + 3 + b, b)
        return carry

    lax.fori_loop(0, NCH // 3 - 1, body, 0)
    j = NCH - 5
    r_wait(0)
    scat(j, 0)
    r_start(j + 3, 0)
    r_wait(1)
    scat(j + 1, 1)
    r_start(j + 4, 1)
    r_wait(2)
    scat(j + 2, 2)
    r_wait(0)
    scat(j + 3, 0)
    r_wait(1)
    scat(j + 4, 1)
    plsc.subcore_barrier()
    pltpu.sync_copy(cacc.at[pl.ds(s * RPT, RPT)],
                    cpart_hbm.at[c].at[pl.ds(s * RPT, RPT)])

    @pl.when(s == NS - 1)
    def _out_tail():
        pltpu.sync_copy(cacc.at[pl.ds(NS * RPT, TAIL)],
                        cpart_hbm.at[c].at[pl.ds(NS * RPT, TAIL)])

    @pl.when(jnp.logical_and(s == 0, c == 0))
    def _out_s0():
        pltpu.sync_copy(sacc, s0_hbm)

    @pl.when(jnp.logical_and(s == 0, c == 1))
    def _out_s1():
        pltpu.sync_copy(sacc, s1_hbm)


def _tc2_body(cp_ref, sp_ref, hv_ref, wtT, bt2, wihT, bih2, whhT, bhh2, out_ref):
    cs = cp_ref[0] + cp_ref[1]
    sv = sp_ref[0, :, 0] + sp_ref[1, :, 0]
    inv = 1.0 / (sv + 1e-16)
    cmean = cs * inv[:, None]
    cfull = jnp.dot(cmean, wtT[...], preferred_element_type=jnp.float32) \
        + (sv * inv)[:, None] * bt2[...]
    ctx = jnp.where(cfull > 0, cfull, jnp.exp(cfull) - 1.0)
    hv = hv_ref[...]
    gi = jnp.dot(ctx, wihT[...], preferred_element_type=jnp.float32) + bih2[...]
    gh = jnp.dot(hv, whhT[...], preferred_element_type=jnp.float32) + bhh2[...]
    r = jax.nn.sigmoid(gi[:, :G] + gh[:, :G])
    zz = jax.nn.sigmoid(gi[:, G:2 * G] + gh[:, G:2 * G])
    nn = jnp.tanh(gi[:, 2 * G:] + r * gh[:, 2 * G:])
    out_ref[...] = jnp.maximum((1.0 - zz) * nn + zz * hv, 0.0)


def _tc2(cpa, sp3, hv, wtT, bt2, wihT, bih2, whhT, bhh2):
    return pl.pallas_call(
        _tc2_body,
        grid=(NNB,),
        in_specs=[
            pl.BlockSpec((NC, NB, G), lambda i: (0, i, 0)),
            pl.BlockSpec((NC, NB, 1), lambda i: (0, i, 0)),
            pl.BlockSpec((NB, G), lambda i: (i, 0)),
            pl.BlockSpec((G, G), lambda i: (0, 0)),
            pl.BlockSpec((1, G), lambda i: (0, 0)),
            pl.BlockSpec((G, 3 * G), lambda i: (0, 0)),
            pl.BlockSpec((1, 3 * G), lambda i: (0, 0)),
            pl.BlockSpec((G, 3 * G), lambda i: (0, 0)),
            pl.BlockSpec((1, 3 * G), lambda i: (0, 0)),
        ],
        out_specs=pl.BlockSpec((NB, G), lambda i: (i, 0)),
        out_shape=jax.ShapeDtypeStruct((N, G), jnp.float32),
    )(cpa, sp3, hv, wtT, bt2, wihT, bih2, whhT, bhh2)


def kernel(x, edge_index, edge_attr, Wn, bn, We1, be1, We2, be2, Wt, bt,
           Wih, bih, Whh, bhh):
    src = edge_index[0]
    dst = edge_index[1]
    wnT = Wn.T
    we1aT = We1[:, :DN].T
    we1bT = We1[:, DN:].T
    w2a = We2[0, :G].reshape(G, 1)
    w2b = We2[0, G:].reshape(G, 1)

    hv, xw1p, snode = _tc0(x, wnT, bn.reshape(1, G), we1aT, be1.reshape(1, G),
                           w2a, be2.reshape(1, 1))

    sn1 = snode.reshape(N)
    zc = jnp.zeros((N, G), jnp.float32)
    zs = jnp.zeros((N,), jnp.float32)

    src2 = src.reshape(NW, NCH, C)
    dst2 = dst.reshape(NW, NCH, C)

    g1, sdst = _sc1(xw1p, sn1, src2, dst2)
    hx, ex3 = _tc1(g1, edge_attr, sdst.reshape(NEB, 1, EB), we1bT, w2b)
    cpart, s0, s1 = _sc2(hx, ex3.reshape(NW * NCH, 1, C), dst2, zc, zs)
    spart = jnp.stack([s0, s1]).reshape(NC, N, 1)

    return _tc2(cpart, spart, hv, Wt.T, bt.reshape(1, G),
                Wih.T, bih.reshape(1, 3 * G), Whh.T, bhh.reshape(1, 3 * G))
